# baseline (speedup 1.0000x reference)
import functools

import jax
import jax.numpy as jnp
from jax import lax
from jax.experimental import pallas as pl
from jax.experimental.pallas import tpu as pltpu
from jax.experimental.pallas import tpu_sc as plsc

PAD = 1
B, T, D = 4096, 200, 64
TP = 224
NW = 32
RPW = B // NW
SEQ = 202
DT = D // 8


def _make_sc_kernel():
    mesh = plsc.VectorSubcoreMesh(core_axis_name="c", subcore_axis_name="s")

    @functools.partial(
        pl.kernel,
        mesh=mesh,
        out_type=jax.ShapeDtypeStruct((T, DT * NW, 8 * RPW), jnp.float32),
        compiler_params=pltpu.CompilerParams(
            needs_layout_passes=False, use_tc_tiling_on_sc=False),
        scratch_types=[
            pltpu.VMEM((RPW * TP,), jnp.int32),
            pltpu.VMEM((TP, RPW + 1), jnp.int32),
            pltpu.VMEM((D, SEQ), jnp.float32),
            pltpu.VMEM((2, D * RPW), jnp.float32),
            pltpu.SemaphoreType.DMA,
            pltpu.SemaphoreType.DMA,
        ],
    )
    def k(tok_hbm, w_hbm, out_hbm, tok_v, idx_tv, w_v, blk_v, s0, s1):
        sems = (s0, s1)
        wid = lax.axis_index("s") * 2 + lax.axis_index("c")
        base = wid * RPW
        lanes = lax.iota(jnp.int32, 16)

        pltpu.sync_copy(w_hbm, w_v)
        pltpu.sync_copy(tok_hbm.at[pl.ds(base * TP, RPW * TP)], tok_v)

        @plsc.parallel_loop(0, RPW, unroll=2)
        def _index_body(rl):
            carry = jnp.int32(0)
            col = jnp.zeros((16,), jnp.int32) + rl
            for c in range(TP // 16):
                t = tok_v[pl.ds(rl * TP + c * 16, 16)]
                m = jnp.where(t != PAD, jnp.int32(1), jnp.int32(0))
                cs = plsc.cumsum(m) + carry
                plsc.store_scatter(idx_tv, [lanes + c * 16, col], cs * m + PAD)
                carry = carry + jnp.sum(m)

        zero16 = jnp.zeros((16,), jnp.int32)

        def compute_block(t, slot):
            ivecs = [idx_tv[t, pl.ds(16 * kk, 16)] for kk in range(8)]

            @plsc.parallel_loop(0, D, unroll=16)
            def _gather(d):
                drow = zero16 + d
                for bc in range(8):
                    vals = plsc.load_gather(w_v, [drow, ivecs[bc]])
                    blk_v[slot, pl.ds(d * 128 + bc * 16, 16)] = vals

            for dt in range(DT):
                pltpu.async_copy(blk_v.at[slot, pl.ds(dt * 8 * RPW, 8 * RPW)],
                                 out_hbm.at[t, dt * NW + wid], sems[slot])

        def drain_block(t, slot):
            for dt in range(DT):
                pltpu.make_async_copy(blk_v.at[slot, pl.ds(dt * 8 * RPW, 8 * RPW)],
                                      out_hbm.at[t, dt * NW + wid],
                                      sems[slot]).wait()

        compute_block(0, 0)
        compute_block(1, 1)

        def body(g, carry_none):
            t0 = 2 * g
            drain_block(t0 - 2, 0)
            compute_block(t0, 0)
            drain_block(t0 - 1, 1)
            compute_block(t0 + 1, 1)
            return carry_none

        lax.fori_loop(1, T // 2, body, 0)
        drain_block(T - 2, 0)
        drain_block(T - 1, 1)

    return k


def kernel(tokens, weight):
    tokens_p = jnp.pad(tokens.astype(jnp.int32), ((0, 0), (0, TP - T)),
                       constant_values=PAD)
    out5 = _make_sc_kernel()(tokens_p.reshape(-1), weight.T)
    out5 = out5.reshape(T, DT, NW, 8, RPW)
    return out5.transpose(2, 4, 0, 1, 3).reshape(B, T, D)


# speedup vs baseline: 1.0389x; 1.0389x over previous
import functools

import jax
import jax.numpy as jnp
from jax import lax
from jax.experimental import pallas as pl
from jax.experimental.pallas import tpu as pltpu
from jax.experimental.pallas import tpu_sc as plsc

PAD = 1
B, T, D = 4096, 200, 64
TP = 224
NW = 32
RPW = B // NW
SEQ = 202
DT = D // 8


def _make_sc_kernel():
    mesh = plsc.VectorSubcoreMesh(core_axis_name="c", subcore_axis_name="s")

    @functools.partial(
        pl.kernel,
        mesh=mesh,
        out_type=jax.ShapeDtypeStruct((T, DT * NW, 8 * RPW), jnp.float32),
        compiler_params=pltpu.CompilerParams(
            needs_layout_passes=False, use_tc_tiling_on_sc=False),
        scratch_types=[
            pltpu.VMEM((RPW * T + 8,), jnp.int32),
            pltpu.VMEM((TP, RPW + 1), jnp.int32),
            pltpu.VMEM((D, SEQ), jnp.float32),
            pltpu.VMEM((2, D * RPW), jnp.float32),
            pltpu.SemaphoreType.DMA,
            pltpu.SemaphoreType.DMA,
        ],
    )
    def k(tok_hbm, w_hbm, out_hbm, tok_v, idx_tv, w_v, blk_v, s0, s1):
        sems = (s0, s1)
        wid = lax.axis_index("s") * 2 + lax.axis_index("c")
        base = wid * RPW
        lanes = lax.iota(jnp.int32, 16)

        pltpu.sync_copy(w_hbm, w_v)
        pltpu.sync_copy(tok_hbm.at[pl.ds(base * T, RPW * T)],
                        tok_v.at[pl.ds(0, RPW * T)])

        @plsc.parallel_loop(0, RPW, unroll=2)
        def _index_body(rl):
            carry = jnp.int32(0)
            col = jnp.zeros((16,), jnp.int32) + rl
            for c in range(13):
                t = tok_v[pl.ds(rl * T + c * 16, 16)]
                ok = t != PAD
                if c == 12:
                    ok = ok & (lanes < 8)
                m = jnp.where(ok, jnp.int32(1), jnp.int32(0))
                cs = plsc.cumsum(m) + carry
                plsc.store_scatter(idx_tv, [lanes + c * 16, col], cs * m + PAD)
                carry = carry + jnp.sum(m)

        zero16 = jnp.zeros((16,), jnp.int32)

        def compute_block(t, slot):
            ivecs = [idx_tv[t, pl.ds(16 * kk, 16)] for kk in range(8)]

            @plsc.parallel_loop(0, D, unroll=8)
            def _gather(d):
                drow = zero16 + d
                for bc in range(8):
                    vals = plsc.load_gather(w_v, [drow, ivecs[bc]])
                    blk_v[slot, pl.ds(d * 128 + bc * 16, 16)] = vals

            for dt in range(DT):
                pltpu.async_copy(blk_v.at[slot, pl.ds(dt * 8 * RPW, 8 * RPW)],
                                 out_hbm.at[t, dt * NW + wid], sems[slot])

        def drain_block(t, slot):
            for dt in range(DT):
                pltpu.make_async_copy(blk_v.at[slot, pl.ds(dt * 8 * RPW, 8 * RPW)],
                                      out_hbm.at[t, dt * NW + wid],
                                      sems[slot]).wait()

        compute_block(0, 0)
        compute_block(1, 1)

        def body(g, carry_none):
            t0 = 2 * g
            drain_block(t0 - 2, 0)
            compute_block(t0, 0)
            drain_block(t0 - 1, 1)
            compute_block(t0 + 1, 1)
            return carry_none

        lax.fori_loop(1, T // 2, body, 0)
        drain_block(T - 2, 0)
        drain_block(T - 1, 1)

    return k


def kernel(tokens, weight):
    out5 = _make_sc_kernel()(tokens.astype(jnp.int32).reshape(-1), weight.T)
    out5 = out5.reshape(T, DT, NW, 8, RPW)
    return out5.transpose(2, 4, 0, 1, 3).reshape(B, T, D)
